# baseline (speedup 1.0000x reference)
import functools

import jax
import jax.numpy as jnp
from jax import lax
from jax.experimental import pallas as pl
from jax.experimental.pallas import tpu as pltpu
from jax.experimental.pallas import tpu_sc as plsc

N = 10000
E = 160000
D = 256
HALF = 128
NC = 2
NS = 16
CH = 128

EPT = E // NS
NFULL = EPT // CH
REM = EPT - NFULL * CH

DPT = E // (NC * NS)
DFULL = DPT // CH
DREM = DPT - DFULL * CH

RPT = N // NS
ZR = 125

WBR = 624
WTAIL = N - NS * WBR

F32 = jnp.float32
_mesh = plsc.VectorSubcoreMesh(core_axis_name="c", subcore_axis_name="s")


def _writeback(sh_ref, out_ref, s):
    pltpu.sync_copy(sh_ref.at[pl.ds(s * WBR, WBR)],
                    out_ref.at[pl.ds(s * WBR, WBR)])

    @pl.when(s == NS - 1)
    def _():
        pltpu.sync_copy(sh_ref.at[pl.ds(NS * WBR, WTAIL)],
                        out_ref.at[pl.ds(NS * WBR, WTAIL)])


def _deg_body(dst_hbm, degp_hbm, zbuf, ones, idx, idx_r, deg_sh):
    c = lax.axis_index("c")
    s = lax.axis_index("s")

    def _zinit(i, carry):
        zbuf[i, :] = jnp.zeros((16,), F32)
        return carry

    def _oinit(i, carry):
        ones[i, :] = jnp.ones((16,), F32)
        return carry

    lax.fori_loop(0, ZR, _zinit, 0)
    lax.fori_loop(0, CH, _oinit, 0)

    for r in range(RPT // ZR):
        pltpu.sync_copy(zbuf, deg_sh.at[pl.ds(s * RPT + r * ZR, ZR)])
    plsc.subcore_barrier()

    base0 = c * (E // NC) + s * DPT

    def _chunk(j, carry):
        b = base0 + j * CH
        pltpu.sync_copy(dst_hbm.at[pl.ds(b, CH)], idx)
        pltpu.sync_copy(ones, deg_sh.at[idx], add=True)
        return carry

    lax.fori_loop(0, DFULL, _chunk, 0)
    b = base0 + DFULL * CH
    pltpu.sync_copy(dst_hbm.at[pl.ds(b, DREM)], idx_r)
    pltpu.sync_copy(ones.at[pl.ds(0, DREM)], deg_sh.at[idx_r], add=True)

    plsc.subcore_barrier()
    _writeback(deg_sh, degp_hbm.at[c], s)


_deg_call = pl.kernel(
    _deg_body,
    out_type=jax.ShapeDtypeStruct((NC, N, 16), F32),
    mesh=_mesh,
    scratch_types=[
        pltpu.VMEM((ZR, 16), F32),
        pltpu.VMEM((CH, 16), F32),
        pltpu.VMEM((CH,), jnp.int32),
        pltpu.VMEM((DREM,), jnp.int32),
        pltpu.VMEM_SHARED((N, 16), F32),
    ],
)


K = 2
NGRP = NFULL // K
ZR2 = 25


def _scat_body(tsB, src_hbm, dst_hbm, accB, zbuf, stage, sidx, didx,
               stage_r, sidx_r, didx_r, sem_i, sem_i2, sem_g, sem_s, acc_sh):
    c = lax.axis_index("c")
    s = lax.axis_index("s")

    def _zinit(q, carry):
        i = q // 8
        k = q % 8
        zbuf[i, pl.ds(k * 16, 16)] = jnp.zeros((16,), F32)
        return carry

    lax.fori_loop(0, ZR2 * 8, _zinit, 0)
    for r in range(RPT // ZR2):
        pltpu.sync_copy(zbuf, acc_sh.at[pl.ds(s * RPT + r * ZR2, ZR2)])
    plsc.subcore_barrier()

    base0 = s * EPT
    myts = tsB.at[c]

    def _group(g, carry):
        dd = []
        for b in range(K):
            @pl.when(g > 0)
            def _():
                pltpu.make_async_copy(
                    stage[b], acc_sh.at[pl.ds(0, CH)], sem_s[b]).wait()

            off = base0 + (g * K + b) * CH
            di = pltpu.async_copy(
                dst_hbm.at[pl.ds(off, CH)], didx[b], sem_i[b])
            ds_ = pltpu.async_copy(
                src_hbm.at[pl.ds(off, CH)], sidx[b], sem_i2[b])
            dd.append((di, ds_))
        dg = []
        for b in range(K):
            dd[b][1].wait()
            dg.append(pltpu.async_copy(myts.at[sidx[b]], stage[b], sem_g[b]))
        for b in range(K):
            dd[b][0].wait()
            dg[b].wait()
            pltpu.async_copy(stage[b], acc_sh.at[didx[b]], sem_s[b], add=True)
        return carry

    lax.fori_loop(0, NGRP, _group, 0)
    for b in range(K):
        pltpu.make_async_copy(stage[b], acc_sh.at[pl.ds(0, CH)], sem_s[b]).wait()

    bt = base0 + NFULL * CH
    pltpu.sync_copy(dst_hbm.at[pl.ds(bt, REM)], didx_r)
    pltpu.sync_copy(src_hbm.at[pl.ds(bt, REM)], sidx_r)
    pltpu.sync_copy(myts.at[sidx_r], stage_r)
    pltpu.sync_copy(stage_r, acc_sh.at[didx_r], add=True)

    plsc.subcore_barrier()
    _writeback(acc_sh, accB.at[c], s)


_scat_call = pl.kernel(
    _scat_body,
    out_type=jax.ShapeDtypeStruct((NC, N, HALF), F32),
    mesh=_mesh,
    scratch_types=[
        pltpu.VMEM((ZR2, HALF), F32),
        [pltpu.VMEM((CH, HALF), F32) for _ in range(K)],
        [pltpu.VMEM((CH,), jnp.int32) for _ in range(K)],
        [pltpu.VMEM((CH,), jnp.int32) for _ in range(K)],
        pltpu.VMEM((REM, HALF), F32),
        pltpu.VMEM((REM,), jnp.int32),
        pltpu.VMEM((REM,), jnp.int32),
        [pltpu.SemaphoreType.DMA for _ in range(K)],
        [pltpu.SemaphoreType.DMA for _ in range(K)],
        [pltpu.SemaphoreType.DMA for _ in range(K)],
        [pltpu.SemaphoreType.DMA for _ in range(K)],
        pltpu.VMEM_SHARED((N, HALF), F32),
    ],
)


BR = 2000
GRID = N // BR

_dot = functools.partial(jnp.dot, preferred_element_type=F32,
                         precision=lax.Precision.HIGHEST)


def _mm_body(x_ref, w_ref, t_ref):
    t_ref[...] = _dot(x_ref[...], w_ref[...])


_mm_call = pl.pallas_call(
    _mm_body,
    grid=(GRID,),
    in_specs=[
        pl.BlockSpec((BR, D), lambda i: (i, 0)),
        pl.BlockSpec((D, D), lambda i: (0, 0)),
    ],
    out_specs=pl.BlockSpec((BR, D), lambda i: (i, 0)),
    out_shape=jax.ShapeDtypeStruct((N, D), F32),
)


def _scale_body(degp_ref, t_ref, dinv_ref, ts_ref):
    deg = degp_ref[0, :, 0:1] + degp_ref[1, :, 0:1] + 1.0
    dinv = lax.rsqrt(deg)
    dinv_ref[...] = dinv
    ts = t_ref[...] * dinv
    ts_ref[...] = jnp.stack([ts[:, :HALF], ts[:, HALF:]], axis=0)


_scale_call = pl.pallas_call(
    _scale_body,
    grid=(GRID,),
    in_specs=[
        pl.BlockSpec((NC, BR, 16), lambda i: (0, i, 0)),
        pl.BlockSpec((BR, D), lambda i: (i, 0)),
    ],
    out_specs=[
        pl.BlockSpec((BR, 1), lambda i: (i, 0)),
        pl.BlockSpec((NC, BR, HALF), lambda i: (0, i, 0)),
    ],
    out_shape=[
        jax.ShapeDtypeStruct((N, 1), F32),
        jax.ShapeDtypeStruct((NC, N, HALF), F32),
    ],
)


def _mid_body(accB_ref, tsB_ref, dinv_ref, b_ref, z_ref, st_ref):
    i = pl.program_id(0)
    acc = jnp.concatenate([accB_ref[0], accB_ref[1]], axis=1)
    ts = jnp.concatenate([tsB_ref[0], tsB_ref[1]], axis=1)
    z = dinv_ref[...] * (acc + ts) + b_ref[...]
    z_ref[...] = z
    st = jnp.stack([jnp.sum(z, axis=0), jnp.sum(z * z, axis=0)], axis=0)

    @pl.when(i == 0)
    def _():
        st_ref[...] = st

    @pl.when(i > 0)
    def _():
        st_ref[...] += st


_mid_call = pl.pallas_call(
    _mid_body,
    grid=(GRID,),
    in_specs=[
        pl.BlockSpec((NC, BR, HALF), lambda i: (0, i, 0)),
        pl.BlockSpec((NC, BR, HALF), lambda i: (0, i, 0)),
        pl.BlockSpec((BR, 1), lambda i: (i, 0)),
        pl.BlockSpec((1, D), lambda i: (0, 0)),
    ],
    out_specs=[
        pl.BlockSpec((BR, D), lambda i: (i, 0)),
        pl.BlockSpec((2, D), lambda i: (0, 0)),
    ],
    out_shape=[
        jax.ShapeDtypeStruct((N, D), F32),
        jax.ShapeDtypeStruct((2, D), F32),
    ],
)


def _norm_relu(z, st, g, be):
    m = st[0:1, :] * (1.0 / N)
    v = st[1:2, :] * (1.0 / N) - m * m
    alpha = g * lax.rsqrt(v + 1e-5)
    beta = be - m * alpha
    return jnp.maximum(z * alpha + beta, 0.0)


def _fuse_body(z_ref, st_ref, g_ref, be_ref, dinv_ref, w_ref, ts_ref):
    y = _norm_relu(z_ref[...], st_ref[...], g_ref[...], be_ref[...])
    ts = _dot(y, w_ref[...]) * dinv_ref[...]
    ts_ref[...] = jnp.stack([ts[:, :HALF], ts[:, HALF:]], axis=0)


_fuse_call = pl.pallas_call(
    _fuse_body,
    grid=(GRID,),
    in_specs=[
        pl.BlockSpec((BR, D), lambda i: (i, 0)),
        pl.BlockSpec((2, D), lambda i: (0, 0)),
        pl.BlockSpec((1, D), lambda i: (0, 0)),
        pl.BlockSpec((1, D), lambda i: (0, 0)),
        pl.BlockSpec((BR, 1), lambda i: (i, 0)),
        pl.BlockSpec((D, D), lambda i: (0, 0)),
    ],
    out_specs=pl.BlockSpec((NC, BR, HALF), lambda i: (0, i, 0)),
    out_shape=jax.ShapeDtypeStruct((NC, N, HALF), F32),
)


def _final_body(z0_ref, st0_ref, g0_ref, be0_ref,
                z1_ref, st1_ref, g1_ref, be1_ref,
                z2_ref, st2_ref, g2_ref, be2_ref,
                w0_ref, w1_ref, w2_ref, fcb_ref, out_ref):
    y0 = _norm_relu(z0_ref[...], st0_ref[...], g0_ref[...], be0_ref[...])
    y1 = _norm_relu(z1_ref[...], st1_ref[...], g1_ref[...], be1_ref[...])
    y2 = _norm_relu(z2_ref[...], st2_ref[...], g2_ref[...], be2_ref[...])
    out = _dot(y0, w0_ref[...]) + _dot(y1, w1_ref[...]) + _dot(y2, w2_ref[...])
    out_ref[...] = out + fcb_ref[...]


_final_call = pl.pallas_call(
    _final_body,
    grid=(GRID,),
    in_specs=(
        [pl.BlockSpec((BR, D), lambda i: (i, 0)),
         pl.BlockSpec((2, D), lambda i: (0, 0)),
         pl.BlockSpec((1, D), lambda i: (0, 0)),
         pl.BlockSpec((1, D), lambda i: (0, 0))] * 3
        + [pl.BlockSpec((D, D), lambda i: (0, 0))] * 3
        + [pl.BlockSpec((1, D), lambda i: (0, 0))]
    ),
    out_specs=pl.BlockSpec((BR, D), lambda i: (i, 0)),
    out_shape=jax.ShapeDtypeStruct((N, D), F32),
)


def kernel(x, edge_index, W0, b0, g0, be0, W1, b1, g1, be1,
           W2, b2, g2, be2, fcW, fcb):
    src = edge_index[0].astype(jnp.int32)
    dst = edge_index[1].astype(jnp.int32)
    row = lambda a: a.reshape(1, D)

    degp = _deg_call(dst)
    t0 = _mm_call(x, W0)
    dinv, ts = _scale_call(degp, t0)

    acc = _scat_call(ts, src, dst)
    z0, st0 = _mid_call(acc, ts, dinv, row(b0))
    ts = _fuse_call(z0, st0, row(g0), row(be0), dinv, W1)

    acc = _scat_call(ts, src, dst)
    z1, st1 = _mid_call(acc, ts, dinv, row(b1))
    ts = _fuse_call(z1, st1, row(g1), row(be1), dinv, W2)

    acc = _scat_call(ts, src, dst)
    z2, st2 = _mid_call(acc, ts, dinv, row(b2))

    return _final_call(z0, st0, row(g0), row(be0),
                       z1, st1, row(g1), row(be1),
                       z2, st2, row(g2), row(be2),
                       fcW[:D], fcW[D:2 * D], fcW[2 * D:], row(fcb))


# speedup vs baseline: 1.0926x; 1.0926x over previous
import functools

import jax
import jax.numpy as jnp
from jax import lax
from jax.experimental import pallas as pl
from jax.experimental.pallas import tpu as pltpu
from jax.experimental.pallas import tpu_sc as plsc

N = 10000
E = 160000
D = 256
HALF = 128
NC = 2
NS = 16
CH = 128

EPT = E // NS
NFULL = EPT // CH
REM = EPT - NFULL * CH

DPT = E // (NC * NS)
DFULL = DPT // CH
DREM = DPT - DFULL * CH

RPT = N // NS
ZR = 125

WBR = 624
WTAIL = N - NS * WBR

F32 = jnp.float32
_mesh = plsc.VectorSubcoreMesh(core_axis_name="c", subcore_axis_name="s")


def _writeback(sh_ref, out_ref, s):
    pltpu.sync_copy(sh_ref.at[pl.ds(s * WBR, WBR)],
                    out_ref.at[pl.ds(s * WBR, WBR)])

    @pl.when(s == NS - 1)
    def _():
        pltpu.sync_copy(sh_ref.at[pl.ds(NS * WBR, WTAIL)],
                        out_ref.at[pl.ds(NS * WBR, WTAIL)])


def _deg_body(dst_hbm, degp_hbm, zbuf, ones, idx, idx_r, deg_sh):
    c = lax.axis_index("c")
    s = lax.axis_index("s")

    def _zinit(i, carry):
        zbuf[i, :] = jnp.zeros((16,), F32)
        return carry

    def _oinit(i, carry):
        ones[i, :] = jnp.ones((16,), F32)
        return carry

    lax.fori_loop(0, ZR, _zinit, 0)
    lax.fori_loop(0, CH, _oinit, 0)

    for r in range(RPT // ZR):
        pltpu.sync_copy(zbuf, deg_sh.at[pl.ds(s * RPT + r * ZR, ZR)])
    plsc.subcore_barrier()

    base0 = c * (E // NC) + s * DPT

    def _chunk(j, carry):
        b = base0 + j * CH
        pltpu.sync_copy(dst_hbm.at[pl.ds(b, CH)], idx)
        pltpu.sync_copy(ones, deg_sh.at[idx], add=True)
        return carry

    lax.fori_loop(0, DFULL, _chunk, 0)
    b = base0 + DFULL * CH
    pltpu.sync_copy(dst_hbm.at[pl.ds(b, DREM)], idx_r)
    pltpu.sync_copy(ones.at[pl.ds(0, DREM)], deg_sh.at[idx_r], add=True)

    plsc.subcore_barrier()
    _writeback(deg_sh, degp_hbm.at[c], s)


_deg_call = pl.kernel(
    _deg_body,
    out_type=jax.ShapeDtypeStruct((NC, N, 16), F32),
    mesh=_mesh,
    scratch_types=[
        pltpu.VMEM((ZR, 16), F32),
        pltpu.VMEM((CH, 16), F32),
        pltpu.VMEM((CH,), jnp.int32),
        pltpu.VMEM((DREM,), jnp.int32),
        pltpu.VMEM_SHARED((N, 16), F32),
    ],
)


K = 3
SCH = 104
SFULL = EPT // SCH
NGRP = SFULL // K
SREM = EPT - SFULL * SCH
ZB = 16


def _scat_body(tsB, src_hbm, dst_hbm, accB, zbuf, stage, sidx, didx,
               sidx_r, didx_r, sem_i, sem_i2, sem_g, sem_s, acc_sh):
    c = lax.axis_index("c")
    s = lax.axis_index("s")

    def _zinit(q, carry):
        i = q // 8
        k = q % 8
        zbuf[i, pl.ds(k * 16, 16)] = jnp.zeros((16,), F32)
        return carry

    lax.fori_loop(0, ZB * 8, _zinit, 0)
    for r in range(RPT // ZB):
        pltpu.sync_copy(zbuf, acc_sh.at[pl.ds(s * RPT + r * ZB, ZB)])
    pltpu.sync_copy(zbuf.at[pl.ds(0, 1)],
                    acc_sh.at[pl.ds(s * RPT + (RPT // ZB) * ZB, 1)])
    plsc.subcore_barrier()

    base0 = s * EPT
    myts = tsB.at[c]

    def _group(g, carry):
        dd = []
        for b in range(K):
            @pl.when(g > 0)
            def _():
                pltpu.make_async_copy(
                    stage[b], acc_sh.at[pl.ds(0, SCH)], sem_s[b]).wait()

            off = base0 + (g * K + b) * SCH
            di = pltpu.async_copy(
                dst_hbm.at[pl.ds(off, SCH)], didx[b], sem_i[b])
            ds_ = pltpu.async_copy(
                src_hbm.at[pl.ds(off, SCH)], sidx[b], sem_i2[b])
            dd.append((di, ds_))
        dg = []
        for b in range(K):
            dd[b][1].wait()
            dg.append(pltpu.async_copy(myts.at[sidx[b]], stage[b], sem_g[b]))
        for b in range(K):
            dd[b][0].wait()
            dg[b].wait()
            pltpu.async_copy(stage[b], acc_sh.at[didx[b]], sem_s[b], add=True)
        return carry

    lax.fori_loop(0, NGRP, _group, 0)
    for b in range(K):
        pltpu.make_async_copy(stage[b], acc_sh.at[pl.ds(0, SCH)], sem_s[b]).wait()

    bt = base0 + SFULL * SCH
    pltpu.sync_copy(dst_hbm.at[pl.ds(bt, SREM)], didx_r)
    pltpu.sync_copy(src_hbm.at[pl.ds(bt, SREM)], sidx_r)
    pltpu.sync_copy(myts.at[sidx_r], zbuf)
    pltpu.sync_copy(zbuf, acc_sh.at[didx_r], add=True)

    plsc.subcore_barrier()
    _writeback(acc_sh, accB.at[c], s)


_scat_call = pl.kernel(
    _scat_body,
    out_type=jax.ShapeDtypeStruct((NC, N, HALF), F32),
    mesh=_mesh,
    scratch_types=[
        pltpu.VMEM((ZB, HALF), F32),
        [pltpu.VMEM((SCH, HALF), F32) for _ in range(K)],
        [pltpu.VMEM((SCH,), jnp.int32) for _ in range(K)],
        [pltpu.VMEM((SCH,), jnp.int32) for _ in range(K)],
        pltpu.VMEM((SREM,), jnp.int32),
        pltpu.VMEM((SREM,), jnp.int32),
        [pltpu.SemaphoreType.DMA for _ in range(K)],
        [pltpu.SemaphoreType.DMA for _ in range(K)],
        [pltpu.SemaphoreType.DMA for _ in range(K)],
        [pltpu.SemaphoreType.DMA for _ in range(K)],
        pltpu.VMEM_SHARED((N, HALF), F32),
    ],
)


BR = 2000
GRID = N // BR

_dot = functools.partial(jnp.dot, preferred_element_type=F32,
                         precision=lax.Precision.HIGHEST)


def _mm_body(x_ref, w_ref, t_ref):
    t_ref[...] = _dot(x_ref[...], w_ref[...])


_mm_call = pl.pallas_call(
    _mm_body,
    grid=(GRID,),
    in_specs=[
        pl.BlockSpec((BR, D), lambda i: (i, 0)),
        pl.BlockSpec((D, D), lambda i: (0, 0)),
    ],
    out_specs=pl.BlockSpec((BR, D), lambda i: (i, 0)),
    out_shape=jax.ShapeDtypeStruct((N, D), F32),
)


def _scale_body(degp_ref, t_ref, dinv_ref, ts_ref):
    deg = degp_ref[0, :, 0:1] + degp_ref[1, :, 0:1] + 1.0
    dinv = lax.rsqrt(deg)
    dinv_ref[...] = dinv
    ts = t_ref[...] * dinv
    ts_ref[...] = jnp.stack([ts[:, :HALF], ts[:, HALF:]], axis=0)


_scale_call = pl.pallas_call(
    _scale_body,
    grid=(GRID,),
    in_specs=[
        pl.BlockSpec((NC, BR, 16), lambda i: (0, i, 0)),
        pl.BlockSpec((BR, D), lambda i: (i, 0)),
    ],
    out_specs=[
        pl.BlockSpec((BR, 1), lambda i: (i, 0)),
        pl.BlockSpec((NC, BR, HALF), lambda i: (0, i, 0)),
    ],
    out_shape=[
        jax.ShapeDtypeStruct((N, 1), F32),
        jax.ShapeDtypeStruct((NC, N, HALF), F32),
    ],
)


def _mid_body(accB_ref, tsB_ref, dinv_ref, b_ref, z_ref, st_ref):
    i = pl.program_id(0)
    acc = jnp.concatenate([accB_ref[0], accB_ref[1]], axis=1)
    ts = jnp.concatenate([tsB_ref[0], tsB_ref[1]], axis=1)
    z = dinv_ref[...] * (acc + ts) + b_ref[...]
    z_ref[...] = z
    st = jnp.stack([jnp.sum(z, axis=0), jnp.sum(z * z, axis=0)], axis=0)

    @pl.when(i == 0)
    def _():
        st_ref[...] = st

    @pl.when(i > 0)
    def _():
        st_ref[...] += st


_mid_call = pl.pallas_call(
    _mid_body,
    grid=(GRID,),
    in_specs=[
        pl.BlockSpec((NC, BR, HALF), lambda i: (0, i, 0)),
        pl.BlockSpec((NC, BR, HALF), lambda i: (0, i, 0)),
        pl.BlockSpec((BR, 1), lambda i: (i, 0)),
        pl.BlockSpec((1, D), lambda i: (0, 0)),
    ],
    out_specs=[
        pl.BlockSpec((BR, D), lambda i: (i, 0)),
        pl.BlockSpec((2, D), lambda i: (0, 0)),
    ],
    out_shape=[
        jax.ShapeDtypeStruct((N, D), F32),
        jax.ShapeDtypeStruct((2, D), F32),
    ],
)


def _norm_relu(z, st, g, be):
    m = st[0:1, :] * (1.0 / N)
    v = st[1:2, :] * (1.0 / N) - m * m
    alpha = g * lax.rsqrt(v + 1e-5)
    beta = be - m * alpha
    return jnp.maximum(z * alpha + beta, 0.0)


def _fuse_body(z_ref, st_ref, g_ref, be_ref, dinv_ref, w_ref, ts_ref):
    y = _norm_relu(z_ref[...], st_ref[...], g_ref[...], be_ref[...])
    ts = _dot(y, w_ref[...]) * dinv_ref[...]
    ts_ref[...] = jnp.stack([ts[:, :HALF], ts[:, HALF:]], axis=0)


_fuse_call = pl.pallas_call(
    _fuse_body,
    grid=(GRID,),
    in_specs=[
        pl.BlockSpec((BR, D), lambda i: (i, 0)),
        pl.BlockSpec((2, D), lambda i: (0, 0)),
        pl.BlockSpec((1, D), lambda i: (0, 0)),
        pl.BlockSpec((1, D), lambda i: (0, 0)),
        pl.BlockSpec((BR, 1), lambda i: (i, 0)),
        pl.BlockSpec((D, D), lambda i: (0, 0)),
    ],
    out_specs=pl.BlockSpec((NC, BR, HALF), lambda i: (0, i, 0)),
    out_shape=jax.ShapeDtypeStruct((NC, N, HALF), F32),
)


def _final_body(z0_ref, st0_ref, g0_ref, be0_ref,
                z1_ref, st1_ref, g1_ref, be1_ref,
                z2_ref, st2_ref, g2_ref, be2_ref,
                w0_ref, w1_ref, w2_ref, fcb_ref, out_ref):
    y0 = _norm_relu(z0_ref[...], st0_ref[...], g0_ref[...], be0_ref[...])
    y1 = _norm_relu(z1_ref[...], st1_ref[...], g1_ref[...], be1_ref[...])
    y2 = _norm_relu(z2_ref[...], st2_ref[...], g2_ref[...], be2_ref[...])
    out = _dot(y0, w0_ref[...]) + _dot(y1, w1_ref[...]) + _dot(y2, w2_ref[...])
    out_ref[...] = out + fcb_ref[...]


_final_call = pl.pallas_call(
    _final_body,
    grid=(GRID,),
    in_specs=(
        [pl.BlockSpec((BR, D), lambda i: (i, 0)),
         pl.BlockSpec((2, D), lambda i: (0, 0)),
         pl.BlockSpec((1, D), lambda i: (0, 0)),
         pl.BlockSpec((1, D), lambda i: (0, 0))] * 3
        + [pl.BlockSpec((D, D), lambda i: (0, 0))] * 3
        + [pl.BlockSpec((1, D), lambda i: (0, 0))]
    ),
    out_specs=pl.BlockSpec((BR, D), lambda i: (i, 0)),
    out_shape=jax.ShapeDtypeStruct((N, D), F32),
)


def kernel(x, edge_index, W0, b0, g0, be0, W1, b1, g1, be1,
           W2, b2, g2, be2, fcW, fcb):
    src = edge_index[0].astype(jnp.int32)
    dst = edge_index[1].astype(jnp.int32)
    row = lambda a: a.reshape(1, D)

    degp = _deg_call(dst)
    t0 = _mm_call(x, W0)
    dinv, ts = _scale_call(degp, t0)

    acc = _scat_call(ts, src, dst)
    z0, st0 = _mid_call(acc, ts, dinv, row(b0))
    ts = _fuse_call(z0, st0, row(g0), row(be0), dinv, W1)

    acc = _scat_call(ts, src, dst)
    z1, st1 = _mid_call(acc, ts, dinv, row(b1))
    ts = _fuse_call(z1, st1, row(g1), row(be1), dinv, W2)

    acc = _scat_call(ts, src, dst)
    z2, st2 = _mid_call(acc, ts, dinv, row(b2))

    return _final_call(z0, st0, row(g0), row(be0),
                       z1, st1, row(g1), row(be1),
                       z2, st2, row(g2), row(be2),
                       fcW[:D], fcW[D:2 * D], fcW[2 * D:], row(fcb))


# speedup vs baseline: 1.1276x; 1.0320x over previous
import functools

import jax
import jax.numpy as jnp
from jax import lax
from jax.experimental import pallas as pl
from jax.experimental.pallas import tpu as pltpu
from jax.experimental.pallas import tpu_sc as plsc

N = 10000
E = 160000
D = 256
HALF = 128
NC = 2
NS = 16
CH = 128

EPT = E // NS
NFULL = EPT // CH
REM = EPT - NFULL * CH

DPT = E // (NC * NS)
DFULL = DPT // CH
DREM = DPT - DFULL * CH

RPT = N // NS
ZR = 125

WBR = 624
WTAIL = N - NS * WBR

F32 = jnp.float32
_mesh = plsc.VectorSubcoreMesh(core_axis_name="c", subcore_axis_name="s")


def _writeback(sh_ref, out_ref, s):
    pltpu.sync_copy(sh_ref.at[pl.ds(s * WBR, WBR)],
                    out_ref.at[pl.ds(s * WBR, WBR)])

    @pl.when(s == NS - 1)
    def _():
        pltpu.sync_copy(sh_ref.at[pl.ds(NS * WBR, WTAIL)],
                        out_ref.at[pl.ds(NS * WBR, WTAIL)])


def _deg_body(dst_hbm, degp_hbm, zbuf, ones, idx, idx_r, sem_i, sem_s, deg_sh):
    c = lax.axis_index("c")
    s = lax.axis_index("s")

    def _zinit(i, carry):
        zbuf[i, :] = jnp.zeros((16,), F32)
        return carry

    def _oinit(i, carry):
        ones[i, :] = jnp.ones((16,), F32)
        return carry

    lax.fori_loop(0, ZR, _zinit, 0)
    lax.fori_loop(0, CH, _oinit, 0)

    for r in range(RPT // ZR):
        pltpu.sync_copy(zbuf, deg_sh.at[pl.ds(s * RPT + r * ZR, ZR)])
    plsc.subcore_barrier()

    base0 = c * (E // NC) + s * DPT

    def _chunk(j, carry):
        b = base0 + j * CH
        pltpu.sync_copy(dst_hbm.at[pl.ds(b, CH)], idx[0])
        pltpu.sync_copy(ones, deg_sh.at[idx[0]], add=True)
        return carry

    lax.fori_loop(0, DFULL, _chunk, 0)
    bt = base0 + DFULL * CH
    pltpu.sync_copy(dst_hbm.at[pl.ds(bt, DREM)], idx_r)
    pltpu.sync_copy(ones.at[pl.ds(0, DREM)], deg_sh.at[idx_r], add=True)

    plsc.subcore_barrier()
    _writeback(deg_sh, degp_hbm.at[c], s)


_deg_call = pl.kernel(
    _deg_body,
    out_type=jax.ShapeDtypeStruct((NC, N, 16), F32),
    mesh=_mesh,
    scratch_types=[
        pltpu.VMEM((ZR, 16), F32),
        pltpu.VMEM((CH, 16), F32),
        [pltpu.VMEM((CH,), jnp.int32) for _ in range(2)],
        pltpu.VMEM((DREM,), jnp.int32),
        [pltpu.SemaphoreType.DMA for _ in range(2)],
        [pltpu.SemaphoreType.DMA for _ in range(2)],
        pltpu.VMEM_SHARED((N, 16), F32),
    ],
)


K = 3
SCH = 104
SFULL = EPT // SCH
NGRP = SFULL // K
SREM = EPT - SFULL * SCH
ZB = 16


def _scat_body(tsB, src_hbm, dst_hbm, accB, zbuf, stage, sidx, didx,
               sidx_r, didx_r, sem_i, sem_i2, sem_g, sem_s, acc_sh):
    c = lax.axis_index("c")
    s = lax.axis_index("s")

    def _zinit(q, carry):
        i = q // 8
        k = q % 8
        zbuf[i, pl.ds(k * 16, 16)] = jnp.zeros((16,), F32)
        return carry

    lax.fori_loop(0, ZB * 8, _zinit, 0)
    for r in range(RPT // ZB):
        pltpu.sync_copy(zbuf, acc_sh.at[pl.ds(s * RPT + r * ZB, ZB)])
    pltpu.sync_copy(zbuf.at[pl.ds(0, 1)],
                    acc_sh.at[pl.ds(s * RPT + (RPT // ZB) * ZB, 1)])
    plsc.subcore_barrier()

    base0 = s * EPT
    myts = tsB.at[c]

    def _group(g, carry):
        dd = []
        for b in range(K):
            @pl.when(g > 0)
            def _():
                pltpu.make_async_copy(
                    stage[b], acc_sh.at[pl.ds(0, SCH)], sem_s[b]).wait()

            off = base0 + (g * K + b) * SCH
            di = pltpu.async_copy(
                dst_hbm.at[pl.ds(off, SCH)], didx[b], sem_i[b])
            ds_ = pltpu.async_copy(
                src_hbm.at[pl.ds(off, SCH)], sidx[b], sem_i2[b])
            dd.append((di, ds_))
        dg = []
        for b in range(K):
            dd[b][1].wait()
            dg.append(pltpu.async_copy(myts.at[sidx[b]], stage[b], sem_g[b]))
        for b in range(K):
            dd[b][0].wait()
            dg[b].wait()
            pltpu.async_copy(stage[b], acc_sh.at[didx[b]], sem_s[b], add=True)
        return carry

    lax.fori_loop(0, NGRP, _group, 0)
    for b in range(K):
        pltpu.make_async_copy(stage[b], acc_sh.at[pl.ds(0, SCH)], sem_s[b]).wait()

    bt = base0 + SFULL * SCH
    pltpu.sync_copy(dst_hbm.at[pl.ds(bt, SREM)], didx_r)
    pltpu.sync_copy(src_hbm.at[pl.ds(bt, SREM)], sidx_r)
    pltpu.sync_copy(myts.at[sidx_r], zbuf)
    pltpu.sync_copy(zbuf, acc_sh.at[didx_r], add=True)

    plsc.subcore_barrier()
    _writeback(acc_sh, accB.at[c], s)


_scat_call = pl.kernel(
    _scat_body,
    out_type=jax.ShapeDtypeStruct((NC, N, HALF), F32),
    mesh=_mesh,
    scratch_types=[
        pltpu.VMEM((ZB, HALF), F32),
        [pltpu.VMEM((SCH, HALF), F32) for _ in range(K)],
        [pltpu.VMEM((SCH,), jnp.int32) for _ in range(K)],
        [pltpu.VMEM((SCH,), jnp.int32) for _ in range(K)],
        pltpu.VMEM((SREM,), jnp.int32),
        pltpu.VMEM((SREM,), jnp.int32),
        [pltpu.SemaphoreType.DMA for _ in range(K)],
        [pltpu.SemaphoreType.DMA for _ in range(K)],
        [pltpu.SemaphoreType.DMA for _ in range(K)],
        [pltpu.SemaphoreType.DMA for _ in range(K)],
        pltpu.VMEM_SHARED((N, HALF), F32),
    ],
)


BR = 2000
GRID = N // BR

_dot = functools.partial(jnp.dot, preferred_element_type=F32,
                         precision=lax.Precision.HIGHEST)


def _mm_body(x_ref, w_ref, t_ref):
    t_ref[...] = _dot(x_ref[...], w_ref[...])


_mm_call = pl.pallas_call(
    _mm_body,
    grid=(GRID,),
    in_specs=[
        pl.BlockSpec((BR, D), lambda i: (i, 0)),
        pl.BlockSpec((D, D), lambda i: (0, 0)),
    ],
    out_specs=pl.BlockSpec((BR, D), lambda i: (i, 0)),
    out_shape=jax.ShapeDtypeStruct((N, D), F32),
)


def _scale_body(degp_ref, t_ref, dinv_ref, ts_ref):
    deg = degp_ref[0, :, 0:1] + degp_ref[1, :, 0:1] + 1.0
    dinv = lax.rsqrt(deg)
    dinv_ref[...] = dinv
    ts = t_ref[...] * dinv
    ts_ref[...] = jnp.stack([ts[:, :HALF], ts[:, HALF:]], axis=0)


_scale_call = pl.pallas_call(
    _scale_body,
    grid=(GRID,),
    in_specs=[
        pl.BlockSpec((NC, BR, 16), lambda i: (0, i, 0)),
        pl.BlockSpec((BR, D), lambda i: (i, 0)),
    ],
    out_specs=[
        pl.BlockSpec((BR, 1), lambda i: (i, 0)),
        pl.BlockSpec((NC, BR, HALF), lambda i: (0, i, 0)),
    ],
    out_shape=[
        jax.ShapeDtypeStruct((N, 1), F32),
        jax.ShapeDtypeStruct((NC, N, HALF), F32),
    ],
)


def _norm_relu(z, st, g, be):
    m = st[0:1, :] * (1.0 / N)
    v = st[1:2, :] * (1.0 / N) - m * m
    alpha = g * lax.rsqrt(v + 1e-5)
    beta = be - m * alpha
    return jnp.maximum(z * alpha + beta, 0.0)


def _phase0(acc_ref, ts_ref, dinv_ref, b_ref, z_sc, st_sc, j):
    acc = jnp.concatenate([acc_ref[0], acc_ref[1]], axis=1)
    ts = jnp.concatenate([ts_ref[0], ts_ref[1]], axis=1)
    z = dinv_ref[...] * (acc + ts) + b_ref[...]
    z_sc[pl.ds(j * BR, BR), :] = z
    st = jnp.stack([jnp.sum(z, axis=0), jnp.sum(z * z, axis=0)], axis=0)

    @pl.when(j == 0)
    def _():
        st_sc[...] = st

    @pl.when(j > 0)
    def _():
        st_sc[...] += st


def _phase1_y(g_ref, be_ref, z_sc, st_sc, j):
    z = z_sc[pl.ds(j * BR, BR), :]
    return _norm_relu(z, st_sc[...], g_ref[...], be_ref[...])


def _mk_layer(first, last):
    def body(acc_ref, ts_ref, dinv_ref, b_ref, g_ref, be_ref, *rest):
        if last:
            fw_ref, fcb_ref = rest[0], rest[1]
            rest = rest[2:]
        else:
            wn_ref, fw_ref = rest[0], rest[1]
            rest = rest[2:]
        if not first:
            fcin_ref = rest[0]
            rest = rest[1:]
        if last:
            out_ref, z_sc, st_sc = rest
        else:
            tsn_ref, out_ref, z_sc, st_sc = rest
        p = pl.program_id(0)
        j = pl.program_id(1)

        @pl.when(p == 0)
        def _():
            _phase0(acc_ref, ts_ref, dinv_ref, b_ref, z_sc, st_sc, j)

        @pl.when(p == 1)
        def _():
            y = _phase1_y(g_ref, be_ref, z_sc, st_sc, j)
            fc = _dot(y, fw_ref[...])
            if not first:
                fc = fc + fcin_ref[...]
            if last:
                out_ref[...] = fc + fcb_ref[...]
            else:
                out_ref[...] = fc
                tsn = _dot(y, wn_ref[...]) * dinv_ref[...]
                tsn_ref[...] = jnp.stack([tsn[:, :HALF], tsn[:, HALF:]], axis=0)

    phase_j = lambda p, j: jnp.where(p == 0, j, 0)
    phase1_j = lambda p, j: jnp.where(p == 1, j, 0)
    in_specs = [
        pl.BlockSpec((NC, BR, HALF), lambda p, j: (0, phase_j(p, j), 0)),
        pl.BlockSpec((NC, BR, HALF), lambda p, j: (0, phase_j(p, j), 0)),
        pl.BlockSpec((BR, 1), lambda p, j: (j, 0)),
        pl.BlockSpec((1, D), lambda p, j: (0, 0)),
        pl.BlockSpec((1, D), lambda p, j: (0, 0)),
        pl.BlockSpec((1, D), lambda p, j: (0, 0)),
        pl.BlockSpec((D, D), lambda p, j: (0, 0)),
        pl.BlockSpec((D, D), lambda p, j: (0, 0)),
    ]
    if last:
        in_specs[7] = pl.BlockSpec((1, D), lambda p, j: (0, 0))
    if not first:
        in_specs.append(pl.BlockSpec((BR, D), lambda p, j: (phase1_j(p, j), 0)))
    out_specs = [pl.BlockSpec((BR, D), lambda p, j: (phase1_j(p, j), 0))]
    out_shape = [jax.ShapeDtypeStruct((N, D), F32)]
    if not last:
        out_specs.insert(0, pl.BlockSpec(
            (NC, BR, HALF), lambda p, j: (0, phase1_j(p, j), 0)))
        out_shape.insert(0, jax.ShapeDtypeStruct((NC, N, HALF), F32))
    return pl.pallas_call(
        body,
        grid=(2, GRID),
        in_specs=in_specs,
        out_specs=out_specs,
        out_shape=out_shape,
        scratch_shapes=[
            pltpu.VMEM((N, D), F32),
            pltpu.VMEM((2, D), F32),
        ],
    )


_layer0_call = _mk_layer(first=True, last=False)
_layer1_call = _mk_layer(first=False, last=False)
_layer2_call = _mk_layer(first=False, last=True)


def kernel(x, edge_index, W0, b0, g0, be0, W1, b1, g1, be1,
           W2, b2, g2, be2, fcW, fcb):
    src = edge_index[0].astype(jnp.int32)
    dst = edge_index[1].astype(jnp.int32)
    row = lambda a: a.reshape(1, D)

    degp = _deg_call(dst)
    t0 = _mm_call(x, W0)
    dinv, ts = _scale_call(degp, t0)

    acc = _scat_call(ts, src, dst)
    ts, fc = _layer0_call(acc, ts, dinv, row(b0), row(g0), row(be0),
                          W1, fcW[:D])
    acc = _scat_call(ts, src, dst)
    ts, fc = _layer1_call(acc, ts, dinv, row(b1), row(g1), row(be1),
                          W2, fcW[D:2 * D], fc)
    acc = _scat_call(ts, src, dst)
    return _layer2_call(acc, ts, dinv, row(b2), row(g2), row(be2),
                        fcW[2 * D:], row(fcb), fc)[0]
